# baseline (speedup 1.0000x reference)
import functools

import jax
import jax.numpy as jnp
from jax import lax
from jax.experimental import pallas as pl
from jax.experimental.pallas import tpu as pltpu
from jax.experimental.pallas import tpu_sc as plsc

N = 10000
E = 320000
D = 128
DH = D // 2
G = 64
OUT = 24
NLAYERS = 7

NC = 2
NT = 16
CHUNK = 128

NCH = 159
EPW = NCH * CHUNK
E_PAD = NT * EPW
NDEPTH = 3
NGRP = NCH // NDEPTH

NW = NC * NT
NCH_DEG = 80
E_PAD_DEG = NW * NCH_DEG * CHUNK

DUMP_ROW = N
AGG_ROWS = 10240
ZERO_ROWS = AGG_ROWS // NT
OUT_ROWS = 640
OUT_ROWS_LAST = N - (NT - 1) * OUT_ROWS
DEG_W = 16
PK_BITS = 14
PK_MASK = (1 << PK_BITS) - 1

RBLK = 2000
GRID = N // RBLK

_mesh = plsc.VectorSubcoreMesh(
    core_axis_name="c", subcore_axis_name="s", num_cores=NC, num_subcores=NT)

_f32 = jnp.float32


def _fill(buf, width, value):
    def row(i, carry):
        for q in range(width // 16):
            buf[i, pl.ds(q * 16, 16)] = jnp.full((16,), value, _f32)
        return carry
    lax.fori_loop(0, CHUNK, row, 0)


def _copy_rows(src, dst, s):
    @pl.when(s < NT - 1)
    def _():
        pltpu.sync_copy(src.at[pl.ds(s * OUT_ROWS, OUT_ROWS)],
                        dst.at[pl.ds(s * OUT_ROWS, OUT_ROWS)])

    @pl.when(s == NT - 1)
    def _():
        pltpu.sync_copy(src.at[pl.ds((NT - 1) * OUT_ROWS, OUT_ROWS_LAST)],
                        dst.at[pl.ds((NT - 1) * OUT_ROWS, OUT_ROWS_LAST)])


def _zero_shared(zbuf, shared, s):
    base = s * ZERO_ROWS
    nfull = ZERO_ROWS // CHUNK
    for k in range(nfull):
        pltpu.sync_copy(zbuf, shared.at[pl.ds(base + k * CHUNK, CHUNK)])
    rem = ZERO_ROWS - nfull * CHUNK
    if rem:
        pltpu.sync_copy(zbuf.at[pl.ds(0, rem)],
                        shared.at[pl.ds(base + nfull * CHUNK, rem)])


@functools.partial(
    pl.kernel,
    out_type=(jax.ShapeDtypeStruct((N, DH), _f32),
              jax.ShapeDtypeStruct((N, DH), _f32)),
    mesh=_mesh,
    scratch_types=(
        pltpu.VMEM((NCH, CHUNK), jnp.int32),
        pltpu.VMEM((1, CHUNK), jnp.int32),
        pltpu.VMEM((1, CHUNK), jnp.int32),
        pltpu.VMEM((1, CHUNK), jnp.int32),
        pltpu.VMEM((1, CHUNK), jnp.int32),
        pltpu.VMEM((1, CHUNK), jnp.int32),
        pltpu.VMEM((1, CHUNK), jnp.int32),
        pltpu.VMEM((CHUNK, DH), _f32),
        pltpu.VMEM((CHUNK, DH), _f32),
        pltpu.VMEM((CHUNK, DH), _f32),
        pltpu.VMEM_SHARED((N, DH), _f32),
        pltpu.VMEM_SHARED((AGG_ROWS, DH), _f32),
        pltpu.SemaphoreType.DMA,
        pltpu.SemaphoreType.DMA,
        pltpu.SemaphoreType.DMA,
        pltpu.SemaphoreType.DMA,
        pltpu.SemaphoreType.DMA,
        pltpu.SemaphoreType.DMA,
    ),
    compiler_params=pltpu.CompilerParams(use_tc_tiling_on_sc=False),
)
def _sc_segsum(m0_hbm, m1_hbm, pkT_hbm, out0, out1,
               pk_v, sb0, sb1, sb2, db0, db1, db2, rb0, rb1, rb2,
               m_sh, agg_sh, g0, g1, g2, s0, s1, s2):
    sbufs = (sb0, sb1, sb2)
    dbufs = (db0, db1, db2)
    rbufs = (rb0, rb1, rb2)
    gsem = (g0, g1, g2)
    ssem = (s0, s1, s2)
    c = lax.axis_index("c")
    s = lax.axis_index("s")

    pltpu.sync_copy(pkT_hbm.at[s], pk_v)

    @pl.when(c == 0)
    def _():
        _copy_rows(m0_hbm, m_sh, s)

    @pl.when(c == 1)
    def _():
        _copy_rows(m1_hbm, m_sh, s)

    _fill(rb0, DH, 0.0)
    _zero_shared(rb0, agg_sh, s)
    plsc.subcore_barrier()

    def _dec(j, srcb, dstb):
        for k in range(CHUNK // 16):
            v = pk_v[j, pl.ds(k * 16, 16)]
            srcb[0, pl.ds(k * 16, 16)] = jnp.bitwise_and(v, PK_MASK)
            dstb[0, pl.ds(k * 16, 16)] = lax.shift_right_logical(v, PK_BITS)

    for b in range(NDEPTH):
        _dec(b, sbufs[b], dbufs[b])
        pltpu.async_copy(m_sh.at[sbufs[b].at[0]], rbufs[b], gsem[b])

    def body(i, carry):
        j = NDEPTH * i
        for b in range(NDEPTH):
            pltpu.make_async_copy(m_sh.at[sbufs[b].at[0]], rbufs[b],
                                  gsem[b]).wait()
            pltpu.async_copy(rbufs[b], agg_sh.at[dbufs[b].at[0]],
                             ssem[b], add=True)

        @pl.when(i < NGRP - 1)
        def _():
            for b in range(NDEPTH):
                pltpu.make_async_copy(rbufs[b], agg_sh.at[dbufs[b].at[0]],
                                      ssem[b]).wait()
                _dec(j + NDEPTH + b, sbufs[b], dbufs[b])
                pltpu.async_copy(m_sh.at[sbufs[b].at[0]], rbufs[b], gsem[b])
        return carry
    lax.fori_loop(0, NGRP, body, 0)

    for b in range(NDEPTH):
        pltpu.make_async_copy(rbufs[b], agg_sh.at[dbufs[b].at[0]],
                              ssem[b]).wait()

    plsc.subcore_barrier()

    @pl.when(c == 0)
    def _():
        _copy_rows(agg_sh, out0, s)

    @pl.when(c == 1)
    def _():
        _copy_rows(agg_sh, out1, s)


@functools.partial(
    pl.kernel,
    out_type=(jax.ShapeDtypeStruct((N, DEG_W), _f32),
              jax.ShapeDtypeStruct((N, DEG_W), _f32)),
    mesh=_mesh,
    scratch_types=(
        pltpu.VMEM((NCH_DEG, CHUNK), jnp.int32),
        pltpu.VMEM((8, CHUNK), jnp.int32),
        pltpu.VMEM((CHUNK, DEG_W), _f32),
        pltpu.VMEM_SHARED((AGG_ROWS, DEG_W), _f32),
        pltpu.SemaphoreType.DMA,
    ),
    compiler_params=pltpu.CompilerParams(use_tc_tiling_on_sc=False),
)
def _sc_degree(pkT_hbm, out0, out1, pk_v, dstg, ob, deg_sh, sd):
    c = lax.axis_index("c")
    s = lax.axis_index("s")
    w = c * NT + s

    pltpu.sync_copy(pkT_hbm.at[w], pk_v)

    _fill(ob, DEG_W, 0.0)
    _zero_shared(ob, deg_sh, s)
    _fill(ob, DEG_W, 1.0)
    plsc.subcore_barrier()

    FIRE = 8

    def body(i, carry):
        j = FIRE * i
        for b in range(FIRE):
            for k in range(CHUNK // 16):
                v = pk_v[j + b, pl.ds(k * 16, 16)]
                dstg[b, pl.ds(k * 16, 16)] = lax.shift_right_logical(v, PK_BITS)
            pltpu.async_copy(ob, deg_sh.at[dstg.at[b]], sd, add=True)
        for b in range(FIRE):
            pltpu.make_async_copy(ob, deg_sh.at[dstg.at[0]], sd).wait()
        return carry
    lax.fori_loop(0, NCH_DEG // FIRE, body, 0)

    plsc.subcore_barrier()

    @pl.when(c == 0)
    def _():
        _copy_rows(deg_sh, out0, s)

    @pl.when(c == 1)
    def _():
        _copy_rows(deg_sh, out1, s)


def _tc_first_body(x_ref, wr_ref, wn_ref, br_ref, m0_ref, m1_ref, r_ref):
    h = x_ref[...]
    m = jnp.dot(h, wr_ref[...], preferred_element_type=_f32,
                precision=lax.Precision.HIGHEST)
    m0_ref[...] = m[:, :DH]
    m1_ref[...] = m[:, DH:]
    r_ref[...] = jnp.dot(h, wn_ref[...], preferred_element_type=_f32,
                         precision=lax.Precision.HIGHEST) + br_ref[...]


_tc_first = pl.pallas_call(
    _tc_first_body,
    grid=(GRID,),
    in_specs=[
        pl.BlockSpec((RBLK, D), lambda i: (i, 0)),
        pl.BlockSpec((D, D), lambda i: (0, 0)),
        pl.BlockSpec((D, D), lambda i: (0, 0)),
        pl.BlockSpec((1, D), lambda i: (0, 0)),
    ],
    out_specs=[
        pl.BlockSpec((RBLK, DH), lambda i: (i, 0)),
        pl.BlockSpec((RBLK, DH), lambda i: (i, 0)),
        pl.BlockSpec((RBLK, D), lambda i: (i, 0)),
    ],
    out_shape=[
        jax.ShapeDtypeStruct((N, DH), _f32),
        jax.ShapeDtypeStruct((N, DH), _f32),
        jax.ShapeDtypeStruct((N, D), _f32),
    ],
)


def _tc_mid_body(a0_ref, a1_ref, rin_ref, d0_ref, d1_ref,
                 wr_ref, wn_ref, br_ref, m0_ref, m1_ref, r_ref):
    den = jnp.maximum(d0_ref[...] + d1_ref[...], 1.0)
    a = jnp.concatenate([a0_ref[...], a1_ref[...]], axis=1)
    h = jnp.maximum(a / den + rin_ref[...], 0.0)
    m = jnp.dot(h, wr_ref[...], preferred_element_type=_f32,
                precision=lax.Precision.HIGHEST)
    m0_ref[...] = m[:, :DH]
    m1_ref[...] = m[:, DH:]
    r_ref[...] = jnp.dot(h, wn_ref[...], preferred_element_type=_f32,
                         precision=lax.Precision.HIGHEST) + br_ref[...]


_tc_mid = pl.pallas_call(
    _tc_mid_body,
    grid=(GRID,),
    in_specs=[
        pl.BlockSpec((RBLK, DH), lambda i: (i, 0)),
        pl.BlockSpec((RBLK, DH), lambda i: (i, 0)),
        pl.BlockSpec((RBLK, D), lambda i: (i, 0)),
        pl.BlockSpec((RBLK, 1), lambda i: (i, 0)),
        pl.BlockSpec((RBLK, 1), lambda i: (i, 0)),
        pl.BlockSpec((D, D), lambda i: (0, 0)),
        pl.BlockSpec((D, D), lambda i: (0, 0)),
        pl.BlockSpec((1, D), lambda i: (0, 0)),
    ],
    out_specs=[
        pl.BlockSpec((RBLK, DH), lambda i: (i, 0)),
        pl.BlockSpec((RBLK, DH), lambda i: (i, 0)),
        pl.BlockSpec((RBLK, D), lambda i: (i, 0)),
    ],
    out_shape=[
        jax.ShapeDtypeStruct((N, DH), _f32),
        jax.ShapeDtypeStruct((N, DH), _f32),
        jax.ShapeDtypeStruct((N, D), _f32),
    ],
)


def _tc_final_body(a0_ref, a1_ref, rin_ref, d0_ref, d1_ref, bat_ref,
                   wo_ref, bo_ref, ps_ref, cnt_ref, out_ref):
    i = pl.program_id(0)
    den = jnp.maximum(d0_ref[...] + d1_ref[...], 1.0)
    a = jnp.concatenate([a0_ref[...], a1_ref[...]], axis=1)
    h = jnp.maximum(a / den + rin_ref[...], 0.0)
    ohT = (lax.broadcasted_iota(jnp.int32, (G, RBLK), 0)
           == bat_ref[0]).astype(_f32)
    part = jnp.dot(ohT, h, preferred_element_type=_f32,
                   precision=lax.Precision.HIGHEST)
    pcnt = jnp.broadcast_to(jnp.sum(ohT, axis=1, keepdims=True), (G, D))

    @pl.when(i == 0)
    def _():
        ps_ref[...] = part
        cnt_ref[...] = pcnt

    @pl.when(i > 0)
    def _():
        ps_ref[...] = ps_ref[...] + part
        cnt_ref[...] = cnt_ref[...] + pcnt

    @pl.when(i == GRID - 1)
    def _():
        pooled = ps_ref[...] / jnp.maximum(cnt_ref[...], 1.0)
        out_ref[...] = jnp.dot(pooled, wo_ref[...], preferred_element_type=_f32,
                               precision=lax.Precision.HIGHEST) + bo_ref[...]


_tc_final = pl.pallas_call(
    _tc_final_body,
    grid=(GRID,),
    in_specs=[
        pl.BlockSpec((RBLK, DH), lambda i: (i, 0)),
        pl.BlockSpec((RBLK, DH), lambda i: (i, 0)),
        pl.BlockSpec((RBLK, D), lambda i: (i, 0)),
        pl.BlockSpec((RBLK, 1), lambda i: (i, 0)),
        pl.BlockSpec((RBLK, 1), lambda i: (i, 0)),
        pl.BlockSpec((1, 1, RBLK), lambda i: (i, 0, 0)),
        pl.BlockSpec((D, D), lambda i: (0, 0)),
        pl.BlockSpec((1, D), lambda i: (0, 0)),
    ],
    out_specs=[
        pl.BlockSpec((G, D), lambda i: (0, 0)),
        pl.BlockSpec((G, D), lambda i: (0, 0)),
        pl.BlockSpec((G, D), lambda i: (0, 0)),
    ],
    out_shape=[
        jax.ShapeDtypeStruct((G, D), _f32),
        jax.ShapeDtypeStruct((G, D), _f32),
        jax.ShapeDtypeStruct((G, D), _f32),
    ],
)


def kernel(x, edge_index, batch, W_rel, b_rel, W_root, W_out, b_out):
    src = edge_index[0]
    dst = edge_index[1]

    def packed(total):
        pad = total - E
        srcP = jnp.concatenate([src, jnp.zeros((pad,), jnp.int32)])
        dstP = jnp.concatenate([dst, jnp.full((pad,), DUMP_ROW, jnp.int32)])
        return (dstP << PK_BITS) | srcP

    pkT = packed(E_PAD).reshape(NT, NCH, CHUNK)
    pkT_deg = packed(E_PAD_DEG).reshape(NW, NCH_DEG, CHUNK)

    d0, d1 = _sc_degree(pkT_deg)
    d0c = d0[:, 0:1]
    d1c = d1[:, 0:1]
    ones_c = jnp.ones((N, 1), _f32)
    zeros_c = jnp.zeros((N, 1), _f32)

    m0, m1, r = _tc_first(x, W_rel[0], W_root[0], b_rel[0][None, :])
    da0, da1 = ones_c, zeros_c
    for i in range(1, NLAYERS):
        a0, a1 = _sc_segsum(m0, m1, pkT)
        m0, m1, r = _tc_mid(a0, a1, r, da0, da1,
                            W_rel[i], W_root[i], b_rel[i][None, :])
        da0, da1 = d0c, d1c
    a0, a1 = _sc_segsum(m0, m1, pkT)

    wo = jnp.pad(W_out, ((0, 0), (0, D - OUT)))
    bo = jnp.pad(b_out, (0, D - OUT))[None, :]
    batT = batch.reshape(GRID, 1, RBLK)
    _ps, _cnt, outp = _tc_final(a0, a1, r, d0c, d1c, batT, wo, bo)
    return outp[:, :OUT]


# speedup vs baseline: 1.1566x; 1.1566x over previous
import functools

import jax
import jax.numpy as jnp
from jax import lax
from jax.experimental import pallas as pl
from jax.experimental.pallas import tpu as pltpu
from jax.experimental.pallas import tpu_sc as plsc

N = 10000
E = 320000
D = 128
DH = D // 2
G = 64
OUT = 24
NLAYERS = 7

NC = 2
NT = 16
CHUNK = 128

NCH = 160
EPW = NCH * CHUNK
E_PAD = NT * EPW
NDEPTH = 2
NGRP = NCH // NDEPTH

NW = NC * NT
NCH_DEG = 80
E_PAD_DEG = NW * NCH_DEG * CHUNK

DUMP_ROW = N
AGG_ROWS = 10240
ZERO_ROWS = AGG_ROWS // NT
OUT_ROWS = 640
OUT_ROWS_LAST = N - (NT - 1) * OUT_ROWS
DEG_W = 16
PK_BITS = 14
PK_MASK = (1 << PK_BITS) - 1

RBLK = 2000
GRID = N // RBLK

_mesh = plsc.VectorSubcoreMesh(
    core_axis_name="c", subcore_axis_name="s", num_cores=NC, num_subcores=NT)

_f32 = jnp.float32


def _fill(buf, width, value):
    def row(i, carry):
        for q in range(width // 16):
            buf[i, pl.ds(q * 16, 16)] = jnp.full((16,), value, _f32)
        return carry
    lax.fori_loop(0, CHUNK, row, 0)


def _copy_rows(src, dst, s):
    @pl.when(s < NT - 1)
    def _():
        pltpu.sync_copy(src.at[pl.ds(s * OUT_ROWS, OUT_ROWS)],
                        dst.at[pl.ds(s * OUT_ROWS, OUT_ROWS)])

    @pl.when(s == NT - 1)
    def _():
        pltpu.sync_copy(src.at[pl.ds((NT - 1) * OUT_ROWS, OUT_ROWS_LAST)],
                        dst.at[pl.ds((NT - 1) * OUT_ROWS, OUT_ROWS_LAST)])


def _zero_shared(zbuf, shared, s):
    base = s * ZERO_ROWS
    nfull = ZERO_ROWS // CHUNK
    for k in range(nfull):
        pltpu.sync_copy(zbuf, shared.at[pl.ds(base + k * CHUNK, CHUNK)])
    rem = ZERO_ROWS - nfull * CHUNK
    if rem:
        pltpu.sync_copy(zbuf.at[pl.ds(0, rem)],
                        shared.at[pl.ds(base + nfull * CHUNK, rem)])


@functools.partial(
    pl.kernel,
    out_type=(jax.ShapeDtypeStruct((N, DH), _f32),
              jax.ShapeDtypeStruct((N, DH), _f32)),
    mesh=_mesh,
    scratch_types=(
        pltpu.VMEM((NCH, CHUNK), jnp.int32),
        pltpu.VMEM((1, CHUNK), jnp.int32),
        pltpu.VMEM((1, CHUNK), jnp.int32),
        pltpu.VMEM((1, CHUNK), jnp.int32),
        pltpu.VMEM((1, CHUNK), jnp.int32),
        pltpu.VMEM((CHUNK, DH), _f32),
        pltpu.VMEM((CHUNK, DH), _f32),
        pltpu.VMEM_SHARED((N, DH), _f32),
        pltpu.VMEM_SHARED((AGG_ROWS, DH), _f32),
        pltpu.SemaphoreType.DMA,
        pltpu.SemaphoreType.DMA,
        pltpu.SemaphoreType.DMA,
        pltpu.SemaphoreType.DMA,
    ),
    compiler_params=pltpu.CompilerParams(use_tc_tiling_on_sc=False),
)
def _sc_segsum(m0_hbm, m1_hbm, pkT_hbm, out0, out1,
               pk_v, sb0, sb1, db0, db1, rb0, rb1,
               m_sh, agg_sh, g0, g1, s0, s1):
    sbufs = (sb0, sb1)
    dbufs = (db0, db1)
    rbufs = (rb0, rb1)
    gsem = (g0, g1)
    ssem = (s0, s1)
    c = lax.axis_index("c")
    s = lax.axis_index("s")

    pltpu.sync_copy(pkT_hbm.at[s], pk_v)

    @pl.when(c == 0)
    def _():
        _copy_rows(m0_hbm, m_sh, s)

    @pl.when(c == 1)
    def _():
        _copy_rows(m1_hbm, m_sh, s)

    _fill(rb0, DH, 0.0)
    _zero_shared(rb0, agg_sh, s)
    plsc.subcore_barrier()

    def _dec(j, srcb, dstb):
        for k in range(CHUNK // 16):
            v = pk_v[j, pl.ds(k * 16, 16)]
            srcb[0, pl.ds(k * 16, 16)] = jnp.bitwise_and(v, PK_MASK)
            dstb[0, pl.ds(k * 16, 16)] = lax.shift_right_logical(v, PK_BITS)

    for b in range(NDEPTH):
        _dec(b, sbufs[b], dbufs[b])
        pltpu.async_copy(m_sh.at[sbufs[b].at[0]], rbufs[b], gsem[b])

    def body(i, carry):
        j = NDEPTH * i
        for b in range(NDEPTH):
            pltpu.make_async_copy(m_sh.at[sbufs[b].at[0]], rbufs[b],
                                  gsem[b]).wait()
            pltpu.async_copy(rbufs[b], agg_sh.at[dbufs[b].at[0]],
                             ssem[b], add=True)

        @pl.when(i < NGRP - 1)
        def _():
            for b in range(NDEPTH):
                pltpu.make_async_copy(rbufs[b], agg_sh.at[dbufs[b].at[0]],
                                      ssem[b]).wait()
                _dec(j + NDEPTH + b, sbufs[b], dbufs[b])
                pltpu.async_copy(m_sh.at[sbufs[b].at[0]], rbufs[b], gsem[b])
        return carry
    lax.fori_loop(0, NGRP, body, 0)

    for b in range(NDEPTH):
        pltpu.make_async_copy(rbufs[b], agg_sh.at[dbufs[b].at[0]],
                              ssem[b]).wait()

    plsc.subcore_barrier()

    @pl.when(c == 0)
    def _():
        _copy_rows(agg_sh, out0, s)

    @pl.when(c == 1)
    def _():
        _copy_rows(agg_sh, out1, s)


@functools.partial(
    pl.kernel,
    out_type=(jax.ShapeDtypeStruct((N, DEG_W), _f32),
              jax.ShapeDtypeStruct((N, DEG_W), _f32)),
    mesh=_mesh,
    scratch_types=(
        pltpu.VMEM((NCH_DEG, CHUNK), jnp.int32),
        pltpu.VMEM((8, CHUNK), jnp.int32),
        pltpu.VMEM((CHUNK, DEG_W), _f32),
        pltpu.VMEM_SHARED((AGG_ROWS, DEG_W), _f32),
        pltpu.SemaphoreType.DMA,
    ),
    compiler_params=pltpu.CompilerParams(use_tc_tiling_on_sc=False),
)
def _sc_degree(pkT_hbm, out0, out1, pk_v, dstg, ob, deg_sh, sd):
    c = lax.axis_index("c")
    s = lax.axis_index("s")
    w = c * NT + s

    pltpu.sync_copy(pkT_hbm.at[w], pk_v)

    _fill(ob, DEG_W, 0.0)
    _zero_shared(ob, deg_sh, s)
    _fill(ob, DEG_W, 1.0)
    plsc.subcore_barrier()

    FIRE = 8

    def body(i, carry):
        j = FIRE * i
        for b in range(FIRE):
            for k in range(CHUNK // 16):
                v = pk_v[j + b, pl.ds(k * 16, 16)]
                dstg[b, pl.ds(k * 16, 16)] = lax.shift_right_logical(v, PK_BITS)
            pltpu.async_copy(ob, deg_sh.at[dstg.at[b]], sd, add=True)
        for b in range(FIRE):
            pltpu.make_async_copy(ob, deg_sh.at[dstg.at[0]], sd).wait()
        return carry
    lax.fori_loop(0, NCH_DEG // FIRE, body, 0)

    plsc.subcore_barrier()

    @pl.when(c == 0)
    def _():
        _copy_rows(deg_sh, out0, s)

    @pl.when(c == 1)
    def _():
        _copy_rows(deg_sh, out1, s)


def _tc_first_body(x_ref, wr_ref, wn_ref, br_ref, m0_ref, m1_ref, r_ref):
    h = x_ref[...]
    m = jnp.dot(h, wr_ref[...], preferred_element_type=_f32,
                precision=lax.Precision.HIGHEST)
    m0_ref[...] = m[:, :DH]
    m1_ref[...] = m[:, DH:]
    r_ref[...] = jnp.dot(h, wn_ref[...], preferred_element_type=_f32,
                         precision=lax.Precision.HIGHEST) + br_ref[...]


_tc_first = pl.pallas_call(
    _tc_first_body,
    grid=(GRID,),
    in_specs=[
        pl.BlockSpec((RBLK, D), lambda i: (i, 0)),
        pl.BlockSpec((D, D), lambda i: (0, 0)),
        pl.BlockSpec((D, D), lambda i: (0, 0)),
        pl.BlockSpec((1, D), lambda i: (0, 0)),
    ],
    out_specs=[
        pl.BlockSpec((RBLK, DH), lambda i: (i, 0)),
        pl.BlockSpec((RBLK, DH), lambda i: (i, 0)),
        pl.BlockSpec((RBLK, D), lambda i: (i, 0)),
    ],
    out_shape=[
        jax.ShapeDtypeStruct((N, DH), _f32),
        jax.ShapeDtypeStruct((N, DH), _f32),
        jax.ShapeDtypeStruct((N, D), _f32),
    ],
)


def _tc_mid_body(a0_ref, a1_ref, rin_ref, d0_ref, d1_ref,
                 wr_ref, wn_ref, br_ref, m0_ref, m1_ref, r_ref):
    den = jnp.maximum(d0_ref[...] + d1_ref[...], 1.0)
    a = jnp.concatenate([a0_ref[...], a1_ref[...]], axis=1)
    h = jnp.maximum(a / den + rin_ref[...], 0.0)
    m = jnp.dot(h, wr_ref[...], preferred_element_type=_f32,
                precision=lax.Precision.HIGHEST)
    m0_ref[...] = m[:, :DH]
    m1_ref[...] = m[:, DH:]
    r_ref[...] = jnp.dot(h, wn_ref[...], preferred_element_type=_f32,
                         precision=lax.Precision.HIGHEST) + br_ref[...]


_tc_mid = pl.pallas_call(
    _tc_mid_body,
    grid=(GRID,),
    in_specs=[
        pl.BlockSpec((RBLK, DH), lambda i: (i, 0)),
        pl.BlockSpec((RBLK, DH), lambda i: (i, 0)),
        pl.BlockSpec((RBLK, D), lambda i: (i, 0)),
        pl.BlockSpec((RBLK, 1), lambda i: (i, 0)),
        pl.BlockSpec((RBLK, 1), lambda i: (i, 0)),
        pl.BlockSpec((D, D), lambda i: (0, 0)),
        pl.BlockSpec((D, D), lambda i: (0, 0)),
        pl.BlockSpec((1, D), lambda i: (0, 0)),
    ],
    out_specs=[
        pl.BlockSpec((RBLK, DH), lambda i: (i, 0)),
        pl.BlockSpec((RBLK, DH), lambda i: (i, 0)),
        pl.BlockSpec((RBLK, D), lambda i: (i, 0)),
    ],
    out_shape=[
        jax.ShapeDtypeStruct((N, DH), _f32),
        jax.ShapeDtypeStruct((N, DH), _f32),
        jax.ShapeDtypeStruct((N, D), _f32),
    ],
)


def _tc_final_body(a0_ref, a1_ref, rin_ref, d0_ref, d1_ref, bat_ref,
                   wo_ref, bo_ref, ps_ref, cnt_ref, out_ref):
    i = pl.program_id(0)
    den = jnp.maximum(d0_ref[...] + d1_ref[...], 1.0)
    a = jnp.concatenate([a0_ref[...], a1_ref[...]], axis=1)
    h = jnp.maximum(a / den + rin_ref[...], 0.0)
    ohT = (lax.broadcasted_iota(jnp.int32, (G, RBLK), 0)
           == bat_ref[0]).astype(_f32)
    part = jnp.dot(ohT, h, preferred_element_type=_f32,
                   precision=lax.Precision.HIGHEST)
    pcnt = jnp.broadcast_to(jnp.sum(ohT, axis=1, keepdims=True), (G, D))

    @pl.when(i == 0)
    def _():
        ps_ref[...] = part
        cnt_ref[...] = pcnt

    @pl.when(i > 0)
    def _():
        ps_ref[...] = ps_ref[...] + part
        cnt_ref[...] = cnt_ref[...] + pcnt

    @pl.when(i == GRID - 1)
    def _():
        pooled = ps_ref[...] / jnp.maximum(cnt_ref[...], 1.0)
        out_ref[...] = jnp.dot(pooled, wo_ref[...], preferred_element_type=_f32,
                               precision=lax.Precision.HIGHEST) + bo_ref[...]


_tc_final = pl.pallas_call(
    _tc_final_body,
    grid=(GRID,),
    in_specs=[
        pl.BlockSpec((RBLK, DH), lambda i: (i, 0)),
        pl.BlockSpec((RBLK, DH), lambda i: (i, 0)),
        pl.BlockSpec((RBLK, D), lambda i: (i, 0)),
        pl.BlockSpec((RBLK, 1), lambda i: (i, 0)),
        pl.BlockSpec((RBLK, 1), lambda i: (i, 0)),
        pl.BlockSpec((1, 1, RBLK), lambda i: (i, 0, 0)),
        pl.BlockSpec((D, D), lambda i: (0, 0)),
        pl.BlockSpec((1, D), lambda i: (0, 0)),
    ],
    out_specs=[
        pl.BlockSpec((G, D), lambda i: (0, 0)),
        pl.BlockSpec((G, D), lambda i: (0, 0)),
        pl.BlockSpec((G, D), lambda i: (0, 0)),
    ],
    out_shape=[
        jax.ShapeDtypeStruct((G, D), _f32),
        jax.ShapeDtypeStruct((G, D), _f32),
        jax.ShapeDtypeStruct((G, D), _f32),
    ],
)


def kernel(x, edge_index, batch, W_rel, b_rel, W_root, W_out, b_out):
    src = edge_index[0]
    dst = edge_index[1]

    def packed(total):
        pad = total - E
        srcP = jnp.concatenate([src, jnp.zeros((pad,), jnp.int32)])
        dstP = jnp.concatenate([dst, jnp.full((pad,), DUMP_ROW, jnp.int32)])
        return (dstP << PK_BITS) | srcP

    pkT = packed(E_PAD).reshape(NT, NCH, CHUNK)
    pkT_deg = packed(E_PAD_DEG).reshape(NW, NCH_DEG, CHUNK)

    d0, d1 = _sc_degree(pkT_deg)
    d0c = d0[:, 0:1]
    d1c = d1[:, 0:1]
    ones_c = jnp.ones((N, 1), _f32)
    zeros_c = jnp.zeros((N, 1), _f32)

    m0, m1, r = _tc_first(x, W_rel[0], W_root[0], b_rel[0][None, :])
    da0, da1 = ones_c, zeros_c
    for i in range(1, NLAYERS):
        a0, a1 = _sc_segsum(m0, m1, pkT)
        m0, m1, r = _tc_mid(a0, a1, r, da0, da1,
                            W_rel[i], W_root[i], b_rel[i][None, :])
        da0, da1 = d0c, d1c
    a0, a1 = _sc_segsum(m0, m1, pkT)

    wo = jnp.pad(W_out, ((0, 0), (0, D - OUT)))
    bo = jnp.pad(b_out, (0, D - OUT))[None, :]
    batT = batch.reshape(GRID, 1, RBLK)
    _ps, _cnt, outp = _tc_final(a0, a1, r, d0c, d1c, batT, wo, bo)
    return outp[:, :OUT]


# speedup vs baseline: 1.1764x; 1.0172x over previous
import functools

import jax
import jax.numpy as jnp
from jax import lax
from jax.experimental import pallas as pl
from jax.experimental.pallas import tpu as pltpu
from jax.experimental.pallas import tpu_sc as plsc

N = 10000
E = 320000
D = 128
DH = D // 2
G = 64
OUT = 24
NLAYERS = 7

NC = 2
NT = 16
CHUNK = 128

NCH = 160
EPW = NCH * CHUNK
E_PAD = NT * EPW
NDEPTH = 2
NGRP = NCH // NDEPTH

NW = NC * NT
NCH_DEG = 80
E_PAD_DEG = NW * NCH_DEG * CHUNK

DUMP_ROW = N
AGG_ROWS = 10240
ZERO_ROWS = AGG_ROWS // NT
OUT_ROWS = 640
OUT_ROWS_LAST = N - (NT - 1) * OUT_ROWS
DEG_W = 16
PK_BITS = 14
PK_MASK = (1 << PK_BITS) - 1

RBLK = 2000
GRID = N // RBLK

_mesh = plsc.VectorSubcoreMesh(
    core_axis_name="c", subcore_axis_name="s", num_cores=NC, num_subcores=NT)

_f32 = jnp.float32


def _fill(buf, width, value):
    def row(i, carry):
        for q in range(width // 16):
            buf[i, pl.ds(q * 16, 16)] = jnp.full((16,), value, _f32)
        return carry
    lax.fori_loop(0, CHUNK, row, 0)


def _copy_rows(src, dst, s):
    @pl.when(s < NT - 1)
    def _():
        pltpu.sync_copy(src.at[pl.ds(s * OUT_ROWS, OUT_ROWS)],
                        dst.at[pl.ds(s * OUT_ROWS, OUT_ROWS)])

    @pl.when(s == NT - 1)
    def _():
        pltpu.sync_copy(src.at[pl.ds((NT - 1) * OUT_ROWS, OUT_ROWS_LAST)],
                        dst.at[pl.ds((NT - 1) * OUT_ROWS, OUT_ROWS_LAST)])


def _copy_rows_async(src, dst, s, sem, wait):
    @pl.when(s < NT - 1)
    def _():
        cp = pltpu.make_async_copy(src.at[pl.ds(s * OUT_ROWS, OUT_ROWS)],
                                   dst.at[pl.ds(s * OUT_ROWS, OUT_ROWS)], sem)
        if wait:
            cp.wait()
        else:
            cp.start()

    @pl.when(s == NT - 1)
    def _():
        cp = pltpu.make_async_copy(
            src.at[pl.ds((NT - 1) * OUT_ROWS, OUT_ROWS_LAST)],
            dst.at[pl.ds((NT - 1) * OUT_ROWS, OUT_ROWS_LAST)], sem)
        if wait:
            cp.wait()
        else:
            cp.start()


def _zero_shared(zbuf, shared, s):
    base = s * ZERO_ROWS
    nfull = ZERO_ROWS // CHUNK
    for k in range(nfull):
        pltpu.sync_copy(zbuf, shared.at[pl.ds(base + k * CHUNK, CHUNK)])
    rem = ZERO_ROWS - nfull * CHUNK
    if rem:
        pltpu.sync_copy(zbuf.at[pl.ds(0, rem)],
                        shared.at[pl.ds(base + nfull * CHUNK, rem)])


@functools.partial(
    pl.kernel,
    out_type=(jax.ShapeDtypeStruct((N, DH), _f32),
              jax.ShapeDtypeStruct((N, DH), _f32)),
    mesh=_mesh,
    scratch_types=(
        pltpu.VMEM((NCH, CHUNK), jnp.int32),
        pltpu.VMEM((1, CHUNK), jnp.int32),
        pltpu.VMEM((1, CHUNK), jnp.int32),
        pltpu.VMEM((1, CHUNK), jnp.int32),
        pltpu.VMEM((1, CHUNK), jnp.int32),
        pltpu.VMEM((CHUNK, DH), _f32),
        pltpu.VMEM((CHUNK, DH), _f32),
        pltpu.VMEM_SHARED((N, DH), _f32),
        pltpu.VMEM_SHARED((AGG_ROWS, DH), _f32),
        pltpu.SemaphoreType.DMA,
        pltpu.SemaphoreType.DMA,
        pltpu.SemaphoreType.DMA,
        pltpu.SemaphoreType.DMA,
    ),
    compiler_params=pltpu.CompilerParams(use_tc_tiling_on_sc=False),
)
def _sc_segsum(m0_hbm, m1_hbm, pkT_hbm, out0, out1,
               pk_v, sb0, sb1, db0, db1, rb0, rb1,
               m_sh, agg_sh, g0, g1, s0, s1):
    sbufs = (sb0, sb1)
    dbufs = (db0, db1)
    rbufs = (rb0, rb1)
    gsem = (g0, g1)
    ssem = (s0, s1)
    c = lax.axis_index("c")
    s = lax.axis_index("s")

    pltpu.async_copy(pkT_hbm.at[s], pk_v, g0)

    @pl.when(c == 0)
    def _():
        _copy_rows_async(m0_hbm, m_sh, s, g1, wait=False)

    @pl.when(c == 1)
    def _():
        _copy_rows_async(m1_hbm, m_sh, s, g1, wait=False)

    _fill(rb0, DH, 0.0)
    _zero_shared(rb0, agg_sh, s)

    pltpu.make_async_copy(pkT_hbm.at[s], pk_v, g0).wait()

    @pl.when(c == 0)
    def _():
        _copy_rows_async(m0_hbm, m_sh, s, g1, wait=True)

    @pl.when(c == 1)
    def _():
        _copy_rows_async(m1_hbm, m_sh, s, g1, wait=True)

    plsc.subcore_barrier()

    def _dec(j, srcb, dstb):
        for k in range(CHUNK // 16):
            v = pk_v[j, pl.ds(k * 16, 16)]
            srcb[0, pl.ds(k * 16, 16)] = jnp.bitwise_and(v, PK_MASK)
            dstb[0, pl.ds(k * 16, 16)] = lax.shift_right_logical(v, PK_BITS)

    for b in range(NDEPTH):
        _dec(b, sbufs[b], dbufs[b])
        pltpu.async_copy(m_sh.at[sbufs[b].at[0]], rbufs[b], gsem[b])

    def body(i, carry):
        j = NDEPTH * i
        for b in range(NDEPTH):
            pltpu.make_async_copy(m_sh.at[sbufs[b].at[0]], rbufs[b],
                                  gsem[b]).wait()
            pltpu.async_copy(rbufs[b], agg_sh.at[dbufs[b].at[0]],
                             ssem[b], add=True)

        @pl.when(i < NGRP - 1)
        def _():
            for b in range(NDEPTH):
                pltpu.make_async_copy(rbufs[b], agg_sh.at[dbufs[b].at[0]],
                                      ssem[b]).wait()
                _dec(j + NDEPTH + b, sbufs[b], dbufs[b])
                pltpu.async_copy(m_sh.at[sbufs[b].at[0]], rbufs[b], gsem[b])
        return carry
    lax.fori_loop(0, NGRP, body, 0)

    for b in range(NDEPTH):
        pltpu.make_async_copy(rbufs[b], agg_sh.at[dbufs[b].at[0]],
                              ssem[b]).wait()

    plsc.subcore_barrier()

    @pl.when(c == 0)
    def _():
        _copy_rows(agg_sh, out0, s)

    @pl.when(c == 1)
    def _():
        _copy_rows(agg_sh, out1, s)


@functools.partial(
    pl.kernel,
    out_type=(jax.ShapeDtypeStruct((N, DEG_W), _f32),
              jax.ShapeDtypeStruct((N, DEG_W), _f32)),
    mesh=_mesh,
    scratch_types=(
        pltpu.VMEM((NCH_DEG, CHUNK), jnp.int32),
        pltpu.VMEM((8, CHUNK), jnp.int32),
        pltpu.VMEM((CHUNK, DEG_W), _f32),
        pltpu.VMEM_SHARED((AGG_ROWS, DEG_W), _f32),
        pltpu.SemaphoreType.DMA,
    ),
    compiler_params=pltpu.CompilerParams(use_tc_tiling_on_sc=False),
)
def _sc_degree(pkT_hbm, out0, out1, pk_v, dstg, ob, deg_sh, sd):
    c = lax.axis_index("c")
    s = lax.axis_index("s")
    w = c * NT + s

    pltpu.sync_copy(pkT_hbm.at[w], pk_v)

    _fill(ob, DEG_W, 0.0)
    _zero_shared(ob, deg_sh, s)
    _fill(ob, DEG_W, 1.0)
    plsc.subcore_barrier()

    FIRE = 8

    def body(i, carry):
        j = FIRE * i
        for b in range(FIRE):
            for k in range(CHUNK // 16):
                v = pk_v[j + b, pl.ds(k * 16, 16)]
                dstg[b, pl.ds(k * 16, 16)] = lax.shift_right_logical(v, PK_BITS)
            pltpu.async_copy(ob, deg_sh.at[dstg.at[b]], sd, add=True)
        for b in range(FIRE):
            pltpu.make_async_copy(ob, deg_sh.at[dstg.at[0]], sd).wait()
        return carry
    lax.fori_loop(0, NCH_DEG // FIRE, body, 0)

    plsc.subcore_barrier()

    @pl.when(c == 0)
    def _():
        _copy_rows(deg_sh, out0, s)

    @pl.when(c == 1)
    def _():
        _copy_rows(deg_sh, out1, s)


def _tc_first_body(x_ref, wr_ref, wn_ref, br_ref, m0_ref, m1_ref, r_ref):
    h = x_ref[...]
    m = jnp.dot(h, wr_ref[...], preferred_element_type=_f32,
                precision=lax.Precision.HIGHEST)
    m0_ref[...] = m[:, :DH]
    m1_ref[...] = m[:, DH:]
    r_ref[...] = jnp.dot(h, wn_ref[...], preferred_element_type=_f32,
                         precision=lax.Precision.HIGHEST) + br_ref[...]


_tc_first = pl.pallas_call(
    _tc_first_body,
    grid=(GRID,),
    in_specs=[
        pl.BlockSpec((RBLK, D), lambda i: (i, 0)),
        pl.BlockSpec((D, D), lambda i: (0, 0)),
        pl.BlockSpec((D, D), lambda i: (0, 0)),
        pl.BlockSpec((1, D), lambda i: (0, 0)),
    ],
    out_specs=[
        pl.BlockSpec((RBLK, DH), lambda i: (i, 0)),
        pl.BlockSpec((RBLK, DH), lambda i: (i, 0)),
        pl.BlockSpec((RBLK, D), lambda i: (i, 0)),
    ],
    out_shape=[
        jax.ShapeDtypeStruct((N, DH), _f32),
        jax.ShapeDtypeStruct((N, DH), _f32),
        jax.ShapeDtypeStruct((N, D), _f32),
    ],
)


def _tc_mid_body(a0_ref, a1_ref, rin_ref, d0_ref, d1_ref,
                 wr_ref, wn_ref, br_ref, m0_ref, m1_ref, r_ref):
    den = jnp.maximum(d0_ref[...] + d1_ref[...], 1.0)
    a = jnp.concatenate([a0_ref[...], a1_ref[...]], axis=1)
    h = jnp.maximum(a / den + rin_ref[...], 0.0)
    m = jnp.dot(h, wr_ref[...], preferred_element_type=_f32,
                precision=lax.Precision.HIGHEST)
    m0_ref[...] = m[:, :DH]
    m1_ref[...] = m[:, DH:]
    r_ref[...] = jnp.dot(h, wn_ref[...], preferred_element_type=_f32,
                         precision=lax.Precision.HIGHEST) + br_ref[...]


_tc_mid = pl.pallas_call(
    _tc_mid_body,
    grid=(GRID,),
    in_specs=[
        pl.BlockSpec((RBLK, DH), lambda i: (i, 0)),
        pl.BlockSpec((RBLK, DH), lambda i: (i, 0)),
        pl.BlockSpec((RBLK, D), lambda i: (i, 0)),
        pl.BlockSpec((RBLK, 1), lambda i: (i, 0)),
        pl.BlockSpec((RBLK, 1), lambda i: (i, 0)),
        pl.BlockSpec((D, D), lambda i: (0, 0)),
        pl.BlockSpec((D, D), lambda i: (0, 0)),
        pl.BlockSpec((1, D), lambda i: (0, 0)),
    ],
    out_specs=[
        pl.BlockSpec((RBLK, DH), lambda i: (i, 0)),
        pl.BlockSpec((RBLK, DH), lambda i: (i, 0)),
        pl.BlockSpec((RBLK, D), lambda i: (i, 0)),
    ],
    out_shape=[
        jax.ShapeDtypeStruct((N, DH), _f32),
        jax.ShapeDtypeStruct((N, DH), _f32),
        jax.ShapeDtypeStruct((N, D), _f32),
    ],
)


def _tc_final_body(a0_ref, a1_ref, rin_ref, d0_ref, d1_ref, bat_ref,
                   wo_ref, bo_ref, ps_ref, cnt_ref, out_ref):
    i = pl.program_id(0)
    den = jnp.maximum(d0_ref[...] + d1_ref[...], 1.0)
    a = jnp.concatenate([a0_ref[...], a1_ref[...]], axis=1)
    h = jnp.maximum(a / den + rin_ref[...], 0.0)
    ohT = (lax.broadcasted_iota(jnp.int32, (G, RBLK), 0)
           == bat_ref[0]).astype(_f32)
    part = jnp.dot(ohT, h, preferred_element_type=_f32,
                   precision=lax.Precision.HIGHEST)
    pcnt = jnp.broadcast_to(jnp.sum(ohT, axis=1, keepdims=True), (G, D))

    @pl.when(i == 0)
    def _():
        ps_ref[...] = part
        cnt_ref[...] = pcnt

    @pl.when(i > 0)
    def _():
        ps_ref[...] = ps_ref[...] + part
        cnt_ref[...] = cnt_ref[...] + pcnt

    @pl.when(i == GRID - 1)
    def _():
        pooled = ps_ref[...] / jnp.maximum(cnt_ref[...], 1.0)
        out_ref[...] = jnp.dot(pooled, wo_ref[...], preferred_element_type=_f32,
                               precision=lax.Precision.HIGHEST) + bo_ref[...]


_tc_final = pl.pallas_call(
    _tc_final_body,
    grid=(GRID,),
    in_specs=[
        pl.BlockSpec((RBLK, DH), lambda i: (i, 0)),
        pl.BlockSpec((RBLK, DH), lambda i: (i, 0)),
        pl.BlockSpec((RBLK, D), lambda i: (i, 0)),
        pl.BlockSpec((RBLK, 1), lambda i: (i, 0)),
        pl.BlockSpec((RBLK, 1), lambda i: (i, 0)),
        pl.BlockSpec((1, 1, RBLK), lambda i: (i, 0, 0)),
        pl.BlockSpec((D, D), lambda i: (0, 0)),
        pl.BlockSpec((1, D), lambda i: (0, 0)),
    ],
    out_specs=[
        pl.BlockSpec((G, D), lambda i: (0, 0)),
        pl.BlockSpec((G, D), lambda i: (0, 0)),
        pl.BlockSpec((G, D), lambda i: (0, 0)),
    ],
    out_shape=[
        jax.ShapeDtypeStruct((G, D), _f32),
        jax.ShapeDtypeStruct((G, D), _f32),
        jax.ShapeDtypeStruct((G, D), _f32),
    ],
)


def kernel(x, edge_index, batch, W_rel, b_rel, W_root, W_out, b_out):
    src = edge_index[0]
    dst = edge_index[1]

    def packed(total):
        pad = total - E
        srcP = jnp.concatenate([src, jnp.zeros((pad,), jnp.int32)])
        dstP = jnp.concatenate([dst, jnp.full((pad,), DUMP_ROW, jnp.int32)])
        return (dstP << PK_BITS) | srcP

    pkT = packed(E_PAD).reshape(NT, NCH, CHUNK)
    pkT_deg = packed(E_PAD_DEG).reshape(NW, NCH_DEG, CHUNK)

    d0, d1 = _sc_degree(pkT_deg)
    d0c = d0[:, 0:1]
    d1c = d1[:, 0:1]
    ones_c = jnp.ones((N, 1), _f32)
    zeros_c = jnp.zeros((N, 1), _f32)

    m0, m1, r = _tc_first(x, W_rel[0], W_root[0], b_rel[0][None, :])
    da0, da1 = ones_c, zeros_c
    for i in range(1, NLAYERS):
        a0, a1 = _sc_segsum(m0, m1, pkT)
        m0, m1, r = _tc_mid(a0, a1, r, da0, da1,
                            W_rel[i], W_root[i], b_rel[i][None, :])
        da0, da1 = d0c, d1c
    a0, a1 = _sc_segsum(m0, m1, pkT)

    wo = jnp.pad(W_out, ((0, 0), (0, D - OUT)))
    bo = jnp.pad(b_out, (0, D - OUT))[None, :]
    batT = batch.reshape(GRID, 1, RBLK)
    _ps, _cnt, outp = _tc_final(a0, a1, r, d0c, d1c, batT, wo, bo)
    return outp[:, :OUT]
